# baseline (device time: 64785 ns/iter reference)
import jax
import jax.numpy as jnp
from jax import lax
from jax.experimental import pallas as pl
from jax.experimental.pallas import tpu as pltpu

N_DEV = 4
B, SQ, DM = 2, 512, 768
HQ, DH = 8, 64
SKV_LOC = 512
WINDOW = 128
O_ROWS = B * SQ
L_ROWS = B * HQ
P_ROWS = O_ROWS + L_ROWS
P_COLS = HQ * DH


def _ring_allreduce_sum(payload):

    def body(p_ref, out_ref, comm_ref, send_sems, recv_sems):
        my = lax.axis_index("i")
        left = (my - 1) % N_DEV
        right = (my + 1) % N_DEV

        barrier_sem = pltpu.get_barrier_semaphore()
        for nbr in (left, right):
            pl.semaphore_signal(
                barrier_sem, inc=1,
                device_id=(nbr,), device_id_type=pl.DeviceIdType.MESH,
            )
        pl.semaphore_wait(barrier_sem, 2)

        out_ref[...] = p_ref[...].astype(jnp.float32)

        for h in range(N_DEV - 1):
            src = p_ref if h == 0 else comm_ref.at[h - 1]
            rdma = pltpu.make_async_remote_copy(
                src_ref=src,
                dst_ref=comm_ref.at[h],
                send_sem=send_sems.at[h],
                recv_sem=recv_sems.at[h],
                device_id=(right,),
                device_id_type=pl.DeviceIdType.MESH,
            )
            rdma.start()
            rdma.wait()
            out_ref[...] += comm_ref[h].astype(jnp.float32)

    return pl.pallas_call(
        body,
        out_shape=jax.ShapeDtypeStruct((P_ROWS, P_COLS), jnp.float32),
        in_specs=[pl.BlockSpec(memory_space=pltpu.VMEM)],
        out_specs=pl.BlockSpec(memory_space=pltpu.VMEM),
        scratch_shapes=[
            pltpu.VMEM((N_DEV - 1, P_ROWS, P_COLS), jnp.bfloat16),
            pltpu.SemaphoreType.DMA((N_DEV - 1,)),
            pltpu.SemaphoreType.DMA((N_DEV - 1,)),
        ],
        compiler_params=pltpu.CompilerParams(collective_id=0),
    )(payload)


def kernel(x, Wq, K_ext, V_ext, Wo):
    my = lax.axis_index("i")

    q2d = jnp.dot(
        x.reshape(B * SQ, DM).astype(jnp.bfloat16),
        Wq.astype(jnp.bfloat16),
        preferred_element_type=jnp.float32,
    )
    Q = (q2d * 0.125).reshape(B, SQ, HQ, DH)

    qi = jnp.arange(SQ)[:, None]
    ki = jnp.arange(SKV_LOC)[None, :] + my * SKV_LOC
    mask = jnp.abs(qi - ki) <= WINDOW

    scores = jnp.einsum(
        "bihd,bjhd->bhij",
        Q.astype(jnp.bfloat16),
        K_ext.astype(jnp.bfloat16),
        preferred_element_type=jnp.float32,
    )
    scores = jnp.where(mask[None, None, :, :], scores, -1e9)
    w = jnp.exp(scores)
    l_part = jnp.sum(w, axis=-1)
    o_part = jnp.einsum(
        "bhij,bjhd->bihd",
        w.astype(jnp.bfloat16),
        V_ext.astype(jnp.bfloat16),
        preferred_element_type=jnp.float32,
    )

    payload = jnp.concatenate(
        [o_part.reshape(O_ROWS, P_COLS), l_part.reshape(L_ROWS, SQ)], axis=0
    ).astype(jnp.bfloat16)

    acc = _ring_allreduce_sum(payload)

    o_sum = acc[:O_ROWS].reshape(B, SQ, HQ, DH)
    l_sum = acc[O_ROWS:].reshape(B, HQ, SQ).transpose(0, 2, 1)
    ctx = (o_sum / l_sum[..., None]).reshape(B, SQ, HQ * DH)

    return jnp.dot(
        ctx.astype(jnp.bfloat16).reshape(B * SQ, HQ * DH),
        Wo.astype(jnp.bfloat16),
        preferred_element_type=jnp.float32,
    ).reshape(B, SQ, DM)


# device time: 49801 ns/iter; 1.3009x vs baseline; 1.3009x over previous
import jax
import jax.numpy as jnp
from jax import lax
from jax.experimental import pallas as pl
from jax.experimental.pallas import tpu as pltpu

N_DEV = 4
B, SQ, DM = 2, 512, 768
HQ, DH = 8, 64
SKV_LOC = 512
WINDOW = 128
O_ROWS = B * SQ
L_ROWS = B * HQ
P_ROWS = O_ROWS + L_ROWS
P_COLS = HQ * DH


def _ring_allreduce_sum(payload):

    def body(p_ref, out_ref, comm_ref, send_sems, recv_sems):
        my = lax.axis_index("i")

        barrier_sem = pltpu.get_barrier_semaphore()
        for d in range(1, N_DEV):
            pl.semaphore_signal(
                barrier_sem, inc=1,
                device_id=((my + d) % N_DEV,),
                device_id_type=pl.DeviceIdType.MESH,
            )
        pl.semaphore_wait(barrier_sem, N_DEV - 1)

        rdmas = []
        for d in range(1, N_DEV):
            rdma = pltpu.make_async_remote_copy(
                src_ref=p_ref,
                dst_ref=comm_ref.at[d - 1],
                send_sem=send_sems.at[d - 1],
                recv_sem=recv_sems.at[d - 1],
                device_id=((my + d) % N_DEV,),
                device_id_type=pl.DeviceIdType.MESH,
            )
            rdma.start()
            rdmas.append(rdma)

        out_ref[...] = p_ref[...].astype(jnp.float32)

        for j in (0, 2, 1):
            rdmas[j].wait_recv()
            out_ref[...] += comm_ref[j].astype(jnp.float32)
        for r in rdmas:
            r.wait_send()

    return pl.pallas_call(
        body,
        out_shape=jax.ShapeDtypeStruct((P_ROWS, P_COLS), jnp.float32),
        in_specs=[pl.BlockSpec(memory_space=pltpu.VMEM)],
        out_specs=pl.BlockSpec(memory_space=pltpu.VMEM),
        scratch_shapes=[
            pltpu.VMEM((N_DEV - 1, P_ROWS, P_COLS), jnp.bfloat16),
            pltpu.SemaphoreType.DMA((N_DEV - 1,)),
            pltpu.SemaphoreType.DMA((N_DEV - 1,)),
        ],
        compiler_params=pltpu.CompilerParams(collective_id=0),
    )(payload)


def kernel(x, Wq, K_ext, V_ext, Wo):
    my = lax.axis_index("i")

    q2d = jnp.dot(
        x.reshape(B * SQ, DM).astype(jnp.bfloat16),
        Wq.astype(jnp.bfloat16),
        preferred_element_type=jnp.float32,
    )
    Q = (q2d * 0.125).reshape(B, SQ, HQ, DH)

    qi = jnp.arange(SQ)[:, None]
    ki = jnp.arange(SKV_LOC)[None, :] + my * SKV_LOC
    mask = jnp.abs(qi - ki) <= WINDOW

    scores = jnp.einsum(
        "bihd,bjhd->bhij",
        Q.astype(jnp.bfloat16),
        K_ext.astype(jnp.bfloat16),
        preferred_element_type=jnp.float32,
    )
    scores = jnp.where(mask[None, None, :, :], scores, -1e9)
    w = jnp.exp(scores)
    l_part = jnp.sum(w, axis=-1)
    o_part = jnp.einsum(
        "bhij,bjhd->bihd",
        w.astype(jnp.bfloat16),
        V_ext.astype(jnp.bfloat16),
        preferred_element_type=jnp.float32,
    )

    payload = jnp.concatenate(
        [o_part.reshape(O_ROWS, P_COLS), l_part.reshape(L_ROWS, SQ)], axis=0
    ).astype(jnp.bfloat16)

    acc = _ring_allreduce_sum(payload)

    o_sum = acc[:O_ROWS].reshape(B, SQ, HQ, DH)
    l_sum = acc[O_ROWS:].reshape(B, HQ, SQ).transpose(0, 2, 1)
    ctx = (o_sum / l_sum[..., None]).reshape(B, SQ, HQ * DH)

    return jnp.dot(
        ctx.astype(jnp.bfloat16).reshape(B * SQ, HQ * DH),
        Wo.astype(jnp.bfloat16),
        preferred_element_type=jnp.float32,
    ).reshape(B, SQ, DM)
